# Initial kernel scaffold; baseline (speedup 1.0000x reference)
import jax
import jax.numpy as jnp
from jax.experimental import pallas as pl


def kernel(S, prototypes, W_proj):
    raise NotImplementedError("write your pallas kernel here")


# speedup vs baseline: 3.0477x; 3.0477x over previous
import functools

import jax
import jax.numpy as jnp
from jax import lax
from jax.experimental import pallas as pl
from jax.experimental.pallas import tpu as pltpu
from jax.experimental.pallas import tpu_sc as plsc

P0 = 200
PP = 256
D = 1024
N = 16384
TOPK = 3
BLK = 512
NEG = -3.0e38
L = 16
NW = 16


def _counts_body(s_ref, ssq_ref, p_ref, psq_ref, out_ref):
    i = pl.program_id(0)
    x = s_ref[...]
    xn = x / (jnp.sqrt(ssq_ref[...][:, 0:1]) + 1e-8)
    pr = p_ref[...]
    pn = pr / (jnp.sqrt(psq_ref[...][:, 0:1]) + 1e-8)
    sim = lax.dot_general(pn, xn, (((1,), (1,)), ((), ())),
                          preferred_element_type=jnp.float32)
    rows = lax.broadcasted_iota(jnp.int32, sim.shape, 0)
    m = jnp.where(rows < P0, sim, NEG)
    onehot = jnp.zeros_like(sim)
    for _ in range(TOPK):
        v = jnp.max(m, axis=0, keepdims=True)
        cand = jnp.where(m == v, rows, PP)
        imin = jnp.min(cand, axis=0, keepdims=True)
        sel = rows == imin
        onehot += sel.astype(jnp.float32)
        m = jnp.where(sel, NEG, m)
    blk = jnp.sum(onehot, axis=1, keepdims=True)

    @pl.when(i == 0)
    def _():
        out_ref[...] = jnp.zeros_like(out_ref)

    out_ref[...] += jnp.broadcast_to(blk, (PP, 8))


def _tail_body(c_ref, cs_ref, p_ref, w_ref, pw_ref, ord_ref, pi_ref):
    pw_ref[...] = jnp.dot(p_ref[...], w_ref[...],
                          preferred_element_type=jnp.float32)
    lanes = lax.broadcasted_iota(jnp.int32, (1, PP), 1)
    ci = c_ref[...].astype(jnp.int32)
    key = jnp.where(lanes < P0, ci * PP + (PP - 1 - lanes), -1 - lanes)

    def body(q, carry):
        ordv, piv = carry
        cq = cs_ref[0, q].astype(jnp.int32)
        kq = jnp.where(q < P0, cq * PP + (PP - 1 - q), -1 - q)
        rq = jnp.sum((key > kq).astype(jnp.int32))
        sel = lanes == rq
        ordv = ordv + jnp.where(sel, q, 0)
        piv = piv + jnp.where(sel, cq.astype(jnp.float32) / 49152.0, 0.0)
        return ordv, piv

    ordv, piv = lax.fori_loop(
        0, PP, body,
        (jnp.zeros((1, PP), jnp.int32), jnp.zeros((1, PP), jnp.float32)))
    ord_ref[...] = jnp.minimum(ordv, P0 - 1)
    pi_ref[...] = piv


@functools.lru_cache(maxsize=1)
def _make_sc_gather():
  @functools.partial(
      pl.kernel,
      mesh=plsc.VectorSubcoreMesh(core_axis_name="c", subcore_axis_name="s"),
      out_type=(
          jax.ShapeDtypeStruct((PP, D), jnp.float32),
          jax.ShapeDtypeStruct((PP, D), jnp.float32),
      ),
      scratch_types=[
          pltpu.VMEM((L,), jnp.int32),
          pltpu.VMEM((L, D), jnp.float32),
          pltpu.VMEM((L, D), jnp.float32),
          pltpu.SemaphoreType.DMA,
      ],
  )
  def _sc_gather(ord_hbm, proto_hbm, pw_hbm,
                 sel_out, pws_out,
                 idx16, rows_a, rows_b, sem):
    wid = lax.axis_index("s") * 2 + lax.axis_index("c")

    @pl.when(wid < NW)
    def _():
      base = wid * L
      pltpu.sync_copy(ord_hbm.at[pl.ds(base, L)], idx16)
      pltpu.async_copy(proto_hbm.at[idx16], rows_a, sem).wait()
      pltpu.async_copy(pw_hbm.at[idx16], rows_b, sem).wait()
      pltpu.sync_copy(rows_a, sel_out.at[pl.ds(base, L)])
      pltpu.sync_copy(rows_b, pws_out.at[pl.ds(base, L)])

  return _sc_gather


def kernel(S, prototypes, W_proj):
    Sq = S[0]
    Ppad = jnp.pad(prototypes, ((0, PP - P0), (0, 0)))
    ssq8 = jnp.broadcast_to(jnp.sum(Sq * Sq, axis=-1, keepdims=True), (N, 8))
    psq8 = jnp.broadcast_to(
        jnp.sum(Ppad * Ppad, axis=-1, keepdims=True), (PP, 8))

    counts_col = pl.pallas_call(
        _counts_body,
        grid=(N // BLK,),
        in_specs=[
            pl.BlockSpec((BLK, D), lambda i: (i, 0)),
            pl.BlockSpec((BLK, 8), lambda i: (i, 0)),
            pl.BlockSpec((PP, D), lambda i: (0, 0)),
            pl.BlockSpec((PP, 8), lambda i: (0, 0)),
        ],
        out_specs=pl.BlockSpec((PP, 8), lambda i: (0, 0)),
        out_shape=jax.ShapeDtypeStruct((PP, 8), jnp.float32),
    )(Sq, ssq8, Ppad, psq8)
    counts = counts_col[:, 0].reshape(1, PP)

    pw, order, pi = pl.pallas_call(
        _tail_body,
        in_specs=[
            pl.BlockSpec((1, PP), lambda: (0, 0)),
            pl.BlockSpec(memory_space=pltpu.SMEM),
            pl.BlockSpec((P0, D), lambda: (0, 0)),
            pl.BlockSpec((D, D), lambda: (0, 0)),
        ],
        out_specs=(
            pl.BlockSpec((P0, D), lambda: (0, 0)),
            pl.BlockSpec((1, PP), lambda: (0, 0)),
            pl.BlockSpec((1, PP), lambda: (0, 0)),
        ),
        out_shape=(
            jax.ShapeDtypeStruct((P0, D), jnp.float32),
            jax.ShapeDtypeStruct((1, PP), jnp.int32),
            jax.ShapeDtypeStruct((1, PP), jnp.float32),
        ),
    )(counts, counts, prototypes, W_proj)

    sel_s, pw_s = _make_sc_gather()(order.reshape(-1), prototypes, pw)

    protos = jnp.concatenate([sel_s[:P0], pw_s[:P0]], axis=-1)
    return jnp.concatenate([pi[0, :P0], protos.reshape(-1)])[None, :]


# speedup vs baseline: 4.1512x; 1.3621x over previous
import functools

import jax
import jax.numpy as jnp
from jax import lax
from jax.experimental import pallas as pl
from jax.experimental.pallas import tpu as pltpu
from jax.experimental.pallas import tpu_sc as plsc

P0 = 200
PP = 256
D = 1024
N = 16384
TOPK = 3
BLK = 512
NEG = -3.0e38
RPW = 8
NW = P0 // RPW


def _counts_body(s_ref, ssq_ref, p_ref, psq_ref, out_ref):
    i = pl.program_id(0)
    x = s_ref[...]
    xn = x / (jnp.sqrt(ssq_ref[...][:, 0:1]) + 1e-8)
    pr = p_ref[...]
    pn = pr / (jnp.sqrt(psq_ref[...][:, 0:1]) + 1e-8)
    sim = lax.dot_general(pn, xn, (((1,), (1,)), ((), ())),
                          preferred_element_type=jnp.float32)
    rows = lax.broadcasted_iota(jnp.int32, sim.shape, 0)
    m = jnp.where(rows < P0, sim, NEG)
    onehot = jnp.zeros_like(sim)
    for _ in range(TOPK):
        v = jnp.max(m, axis=0, keepdims=True)
        cand = jnp.where(m == v, rows, PP)
        imin = jnp.min(cand, axis=0, keepdims=True)
        sel = rows == imin
        onehot += sel.astype(jnp.float32)
        m = jnp.where(sel, NEG, m)
    blk = jnp.sum(onehot, axis=1, keepdims=True)

    @pl.when(i == 0)
    def _():
        out_ref[...] = jnp.zeros_like(out_ref)

    out_ref[...] += jnp.broadcast_to(blk, (PP, 8))


def _tail_body(c_ref, p_ref, w_ref, pw_ref, ord_ref, pi_ref):
    pw_ref[...] = jnp.dot(p_ref[...], w_ref[...],
                          preferred_element_type=jnp.float32)
    ccol = c_ref[...][:, 0:1]
    rows = lax.broadcasted_iota(jnp.int32, (PP, PP), 0)
    lanes = lax.broadcasted_iota(jnp.int32, (PP, PP), 1)
    diagd = jnp.where(rows == lanes, jnp.broadcast_to(ccol, (PP, PP)), 0.0)
    crow = lax.dot_general(jnp.ones((1, PP), jnp.float32), diagd,
                           (((1,), (0,)), ((), ())),
                           precision=lax.Precision.HIGHEST,
                           preferred_element_type=jnp.float32)
    ricol = lax.broadcasted_iota(jnp.int32, (PP, 1), 0)
    lirow = lax.broadcasted_iota(jnp.int32, (1, PP), 1)
    kcol = jnp.where(ricol < P0,
                     ccol.astype(jnp.int32) * PP + (PP - 1 - ricol),
                     -1 - ricol)
    krow = jnp.where(lirow < P0,
                     crow.astype(jnp.int32) * PP + (PP - 1 - lirow),
                     -1 - lirow)
    gt = (kcol > krow).astype(jnp.float32)
    rank = jnp.sum(gt, axis=0, keepdims=True)
    pm = (jnp.broadcast_to(rank.astype(jnp.int32), (PP, PP)) == rows)
    pmf = pm.astype(jnp.float32)
    ordv = jnp.sum(pmf * lanes.astype(jnp.float32), axis=1, keepdims=True)
    ord_i = jnp.minimum(ordv.astype(jnp.int32), P0 - 1)
    piv = jnp.sum(pmf * (crow / 49152.0), axis=1, keepdims=True)
    ord_ref[...] = jnp.broadcast_to(ord_i, (PP, 8))
    pi_ref[...] = jnp.broadcast_to(piv, (PP, 8))


@functools.lru_cache(maxsize=1)
def _make_sc_gather():
  @functools.partial(
      pl.kernel,
      mesh=plsc.VectorSubcoreMesh(core_axis_name="c", subcore_axis_name="s"),
      out_type=jax.ShapeDtypeStruct((P0, 2 * D), jnp.float32),
      scratch_types=[
          pltpu.VMEM((RPW,), jnp.int32),
          pltpu.VMEM((RPW, D), jnp.float32),
          pltpu.VMEM((RPW, D), jnp.float32),
          pltpu.SemaphoreType.DMA,
      ],
  )
  def _sc_gather(ord_hbm, proto_hbm, pw_hbm, out_hbm,
                 idx8, rows_a, rows_b, sem):
    wid = lax.axis_index("s") * 2 + lax.axis_index("c")

    @pl.when(wid < NW)
    def _():
      base = wid * RPW
      pltpu.sync_copy(ord_hbm.at[pl.ds(base, RPW)], idx8)
      ca = pltpu.async_copy(proto_hbm.at[idx8], rows_a, sem)
      cb = pltpu.async_copy(pw_hbm.at[idx8], rows_b, sem)
      ca.wait()
      cb.wait()
      pltpu.sync_copy(rows_a, out_hbm.at[pl.ds(base, RPW), pl.ds(0, D)])
      pltpu.sync_copy(rows_b, out_hbm.at[pl.ds(base, RPW), pl.ds(D, D)])

  return _sc_gather


def kernel(S, prototypes, W_proj):
    Sq = S[0]
    Ppad = jnp.pad(prototypes, ((0, PP - P0), (0, 0)))
    ssq8 = jnp.broadcast_to(jnp.sum(Sq * Sq, axis=-1, keepdims=True), (N, 8))
    psq8 = jnp.broadcast_to(
        jnp.sum(Ppad * Ppad, axis=-1, keepdims=True), (PP, 8))

    counts_col = pl.pallas_call(
        _counts_body,
        grid=(N // BLK,),
        in_specs=[
            pl.BlockSpec((BLK, D), lambda i: (i, 0)),
            pl.BlockSpec((BLK, 8), lambda i: (i, 0)),
            pl.BlockSpec((PP, D), lambda i: (0, 0)),
            pl.BlockSpec((PP, 8), lambda i: (0, 0)),
        ],
        out_specs=pl.BlockSpec((PP, 8), lambda i: (0, 0)),
        out_shape=jax.ShapeDtypeStruct((PP, 8), jnp.float32),
    )(Sq, ssq8, Ppad, psq8)

    pw, ord8, pi8 = pl.pallas_call(
        _tail_body,
        in_specs=[
            pl.BlockSpec((PP, 8), lambda: (0, 0)),
            pl.BlockSpec((P0, D), lambda: (0, 0)),
            pl.BlockSpec((D, D), lambda: (0, 0)),
        ],
        out_specs=(
            pl.BlockSpec((P0, D), lambda: (0, 0)),
            pl.BlockSpec((PP, 8), lambda: (0, 0)),
            pl.BlockSpec((PP, 8), lambda: (0, 0)),
        ),
        out_shape=(
            jax.ShapeDtypeStruct((P0, D), jnp.float32),
            jax.ShapeDtypeStruct((PP, 8), jnp.int32),
            jax.ShapeDtypeStruct((PP, 8), jnp.float32),
        ),
    )(counts_col, prototypes, W_proj)

    out2d = _make_sc_gather()(ord8[:, 0], prototypes, pw)
    return jnp.concatenate([pi8[:P0, 0], out2d.reshape(-1)])[None, :]


# speedup vs baseline: 4.1605x; 1.0023x over previous
import functools

import jax
import jax.numpy as jnp
from jax import lax
from jax.experimental import pallas as pl
from jax.experimental.pallas import tpu as pltpu
from jax.experimental.pallas import tpu_sc as plsc

P0 = 200
PP = 256
D = 1024
N = 16384
TOPK = 3
BLK = 512
NEG = -3.0e38
RPW = 8
NW = P0 // RPW


def _counts_body(s_ref, ssq_ref, p_ref, psq_ref, out_ref, pn_ref):
    i = pl.program_id(0)

    @pl.when(i == 0)
    def _():
        pr = p_ref[...]
        pn_ref[...] = pr / (jnp.sqrt(psq_ref[...][:, 0:1]) + 1e-8)

    x = s_ref[...]
    xn = x / (jnp.sqrt(ssq_ref[...][:, 0:1]) + 1e-8)
    pn = pn_ref[...]
    sim = lax.dot_general(pn, xn, (((1,), (1,)), ((), ())),
                          preferred_element_type=jnp.float32)
    rows = lax.broadcasted_iota(jnp.int32, sim.shape, 0)
    m = jnp.where(rows < P0, sim, NEG)
    onehot = jnp.zeros_like(sim)
    for _ in range(TOPK):
        v = jnp.max(m, axis=0, keepdims=True)
        cand = jnp.where(m == v, rows, PP)
        imin = jnp.min(cand, axis=0, keepdims=True)
        sel = rows == imin
        onehot += sel.astype(jnp.float32)
        m = jnp.where(sel, NEG, m)
    blk = jnp.sum(onehot, axis=1, keepdims=True)

    @pl.when(i == 0)
    def _():
        out_ref[...] = jnp.zeros_like(out_ref)

    out_ref[...] += jnp.broadcast_to(blk, (PP, 8))


def _tail_body(c_ref, p_ref, w_ref, pw_ref, ord_ref, pi_ref):
    pw_ref[...] = jnp.dot(p_ref[...], w_ref[...],
                          preferred_element_type=jnp.float32)
    ccol = c_ref[...][:, 0:1]
    rows = lax.broadcasted_iota(jnp.int32, (PP, PP), 0)
    lanes = lax.broadcasted_iota(jnp.int32, (PP, PP), 1)
    diagd = jnp.where(rows == lanes, jnp.broadcast_to(ccol, (PP, PP)), 0.0)
    crow = lax.dot_general(jnp.ones((1, PP), jnp.float32), diagd,
                           (((1,), (0,)), ((), ())),
                           precision=lax.Precision.HIGHEST,
                           preferred_element_type=jnp.float32)
    ricol = lax.broadcasted_iota(jnp.int32, (PP, 1), 0)
    lirow = lax.broadcasted_iota(jnp.int32, (1, PP), 1)
    kcol = jnp.where(ricol < P0,
                     ccol.astype(jnp.int32) * PP + (PP - 1 - ricol),
                     -1 - ricol)
    krow = jnp.where(lirow < P0,
                     crow.astype(jnp.int32) * PP + (PP - 1 - lirow),
                     -1 - lirow)
    gt = (kcol > krow).astype(jnp.float32)
    rank = jnp.sum(gt, axis=0, keepdims=True)
    pm = (jnp.broadcast_to(rank.astype(jnp.int32), (PP, PP)) == rows)
    pmf = pm.astype(jnp.float32)
    ordv = jnp.sum(pmf * lanes.astype(jnp.float32), axis=1, keepdims=True)
    ord_i = jnp.minimum(ordv.astype(jnp.int32), P0 - 1)
    piv = jnp.sum(pmf * (crow / 49152.0), axis=1, keepdims=True)
    ord_ref[...] = jnp.broadcast_to(ord_i, (PP, 8))
    pi_ref[...] = jnp.broadcast_to(piv, (PP, 8))


@functools.lru_cache(maxsize=1)
def _make_sc_gather():
  @functools.partial(
      pl.kernel,
      mesh=plsc.VectorSubcoreMesh(core_axis_name="c", subcore_axis_name="s"),
      out_type=jax.ShapeDtypeStruct((P0, 2 * D), jnp.float32),
      scratch_types=[
          pltpu.VMEM((RPW,), jnp.int32),
          pltpu.VMEM((RPW, D), jnp.float32),
          pltpu.VMEM((RPW, D), jnp.float32),
          pltpu.SemaphoreType.DMA,
      ],
  )
  def _sc_gather(ord_hbm, proto_hbm, pw_hbm, out_hbm,
                 idx8, rows_a, rows_b, sem):
    wid = lax.axis_index("s") * 2 + lax.axis_index("c")

    @pl.when(wid < NW)
    def _():
      base = wid * RPW
      pltpu.sync_copy(ord_hbm.at[pl.ds(base, RPW)], idx8)
      ca = pltpu.async_copy(proto_hbm.at[idx8], rows_a, sem)
      cb = pltpu.async_copy(pw_hbm.at[idx8], rows_b, sem)
      ca.wait()
      cb.wait()
      pltpu.sync_copy(rows_a, out_hbm.at[pl.ds(base, RPW), pl.ds(0, D)])
      pltpu.sync_copy(rows_b, out_hbm.at[pl.ds(base, RPW), pl.ds(D, D)])

  return _sc_gather


def kernel(S, prototypes, W_proj):
    Sq = S[0]
    Ppad = jnp.pad(prototypes, ((0, PP - P0), (0, 0)))
    ssq8 = jnp.broadcast_to(jnp.sum(Sq * Sq, axis=-1, keepdims=True), (N, 8))
    psq8 = jnp.broadcast_to(
        jnp.sum(Ppad * Ppad, axis=-1, keepdims=True), (PP, 8))

    counts_col = pl.pallas_call(
        _counts_body,
        grid=(N // BLK,),
        in_specs=[
            pl.BlockSpec((BLK, D), lambda i: (i, 0)),
            pl.BlockSpec((BLK, 8), lambda i: (i, 0)),
            pl.BlockSpec((PP, D), lambda i: (0, 0)),
            pl.BlockSpec((PP, 8), lambda i: (0, 0)),
        ],
        out_specs=pl.BlockSpec((PP, 8), lambda i: (0, 0)),
        out_shape=jax.ShapeDtypeStruct((PP, 8), jnp.float32),
        scratch_shapes=[pltpu.VMEM((PP, D), jnp.float32)],
    )(Sq, ssq8, Ppad, psq8)

    pw, ord8, pi8 = pl.pallas_call(
        _tail_body,
        in_specs=[
            pl.BlockSpec((PP, 8), lambda: (0, 0)),
            pl.BlockSpec((P0, D), lambda: (0, 0)),
            pl.BlockSpec((D, D), lambda: (0, 0)),
        ],
        out_specs=(
            pl.BlockSpec((P0, D), lambda: (0, 0)),
            pl.BlockSpec((PP, 8), lambda: (0, 0)),
            pl.BlockSpec((PP, 8), lambda: (0, 0)),
        ),
        out_shape=(
            jax.ShapeDtypeStruct((P0, D), jnp.float32),
            jax.ShapeDtypeStruct((PP, 8), jnp.int32),
            jax.ShapeDtypeStruct((PP, 8), jnp.float32),
        ),
    )(counts_col, prototypes, W_proj)

    out2d = _make_sc_gather()(ord8[:, 0], prototypes, pw)
    return jnp.concatenate([pi8[:P0, 0], out2d.reshape(-1)])[None, :]


# speedup vs baseline: 4.6641x; 1.1210x over previous
import functools

import jax
import jax.numpy as jnp
from jax import lax
from jax.experimental import pallas as pl
from jax.experimental.pallas import tpu as pltpu
from jax.experimental.pallas import tpu_sc as plsc

P0 = 200
PP = 256
D = 1024
N = 16384
TOPK = 3
BLK = 1024
NEG = -3.0e38
RPW = 8
NW = P0 // RPW


def _counts_body(s_ref, ssq_ref, p_ref, psq_ref, out_ref, pn_ref):
    i = pl.program_id(0)

    @pl.when(i == 0)
    def _():
        pr = p_ref[...]
        pn_ref[...] = pr / (jnp.sqrt(psq_ref[...][:, 0:1]) + 1e-8)

    x = s_ref[...]
    xn = x / (jnp.sqrt(ssq_ref[...][:, 0:1]) + 1e-8)
    pn = pn_ref[...]
    sim = lax.dot_general(pn, xn, (((1,), (1,)), ((), ())),
                          preferred_element_type=jnp.float32)
    rows = lax.broadcasted_iota(jnp.int32, sim.shape, 0)
    m = jnp.where(rows < P0, sim, NEG)
    onehot = jnp.zeros_like(sim)
    for _ in range(TOPK):
        v = jnp.max(m, axis=0, keepdims=True)
        cand = jnp.where(m == v, rows, PP)
        imin = jnp.min(cand, axis=0, keepdims=True)
        sel = rows == imin
        onehot += sel.astype(jnp.float32)
        m = jnp.where(sel, NEG, m)
    blk = jnp.sum(onehot, axis=1, keepdims=True)

    @pl.when(i == 0)
    def _():
        out_ref[...] = jnp.zeros_like(out_ref)

    out_ref[...] += jnp.broadcast_to(blk, (PP, 8))


def _tail_body(c_ref, p_ref, w_ref, pw_ref, ord_ref, pi_ref):
    pw_ref[...] = jnp.dot(p_ref[...], w_ref[...],
                          preferred_element_type=jnp.float32)
    ccol = c_ref[...][:, 0:1]
    rows = lax.broadcasted_iota(jnp.int32, (PP, PP), 0)
    lanes = lax.broadcasted_iota(jnp.int32, (PP, PP), 1)
    diagd = jnp.where(rows == lanes, jnp.broadcast_to(ccol, (PP, PP)), 0.0)
    crow = lax.dot_general(jnp.ones((1, PP), jnp.float32), diagd,
                           (((1,), (0,)), ((), ())),
                           precision=lax.Precision.HIGHEST,
                           preferred_element_type=jnp.float32)
    ricol = lax.broadcasted_iota(jnp.int32, (PP, 1), 0)
    lirow = lax.broadcasted_iota(jnp.int32, (1, PP), 1)
    kcol = jnp.where(ricol < P0,
                     ccol.astype(jnp.int32) * PP + (PP - 1 - ricol),
                     -1 - ricol)
    krow = jnp.where(lirow < P0,
                     crow.astype(jnp.int32) * PP + (PP - 1 - lirow),
                     -1 - lirow)
    gt = (kcol > krow).astype(jnp.float32)
    rank = jnp.sum(gt, axis=0, keepdims=True)
    pm = (jnp.broadcast_to(rank.astype(jnp.int32), (PP, PP)) == rows)
    pmf = pm.astype(jnp.float32)
    ordv = jnp.sum(pmf * lanes.astype(jnp.float32), axis=1, keepdims=True)
    ord_i = jnp.minimum(ordv.astype(jnp.int32), P0 - 1)
    piv = jnp.sum(pmf * (crow / 49152.0), axis=1, keepdims=True)
    ord_ref[...] = jnp.broadcast_to(ord_i, (PP, 8))
    pi_ref[...] = jnp.broadcast_to(piv, (PP, 8))


@functools.lru_cache(maxsize=1)
def _make_sc_gather():
  @functools.partial(
      pl.kernel,
      mesh=plsc.VectorSubcoreMesh(core_axis_name="c", subcore_axis_name="s"),
      out_type=jax.ShapeDtypeStruct((P0, 2 * D), jnp.float32),
      scratch_types=[
          pltpu.VMEM((RPW,), jnp.int32),
          pltpu.VMEM((RPW, D), jnp.float32),
          pltpu.VMEM((RPW, D), jnp.float32),
          pltpu.SemaphoreType.DMA,
      ],
  )
  def _sc_gather(ord_hbm, proto_hbm, pw_hbm, out_hbm,
                 idx8, rows_a, rows_b, sem):
    wid = lax.axis_index("s") * 2 + lax.axis_index("c")

    @pl.when(wid < NW)
    def _():
      base = wid * RPW
      pltpu.sync_copy(ord_hbm.at[pl.ds(base, RPW)], idx8)
      ca = pltpu.async_copy(proto_hbm.at[idx8], rows_a, sem)
      cb = pltpu.async_copy(pw_hbm.at[idx8], rows_b, sem)
      ca.wait()
      cb.wait()
      pltpu.sync_copy(rows_a, out_hbm.at[pl.ds(base, RPW), pl.ds(0, D)])
      pltpu.sync_copy(rows_b, out_hbm.at[pl.ds(base, RPW), pl.ds(D, D)])

  return _sc_gather


def kernel(S, prototypes, W_proj):
    Sq = S[0]
    Ppad = jnp.pad(prototypes, ((0, PP - P0), (0, 0)))
    ssq8 = jnp.broadcast_to(jnp.sum(Sq * Sq, axis=-1, keepdims=True), (N, 8))
    psq8 = jnp.broadcast_to(
        jnp.sum(Ppad * Ppad, axis=-1, keepdims=True), (PP, 8))

    counts_col = pl.pallas_call(
        _counts_body,
        grid=(N // BLK,),
        in_specs=[
            pl.BlockSpec((BLK, D), lambda i: (i, 0)),
            pl.BlockSpec((BLK, 8), lambda i: (i, 0)),
            pl.BlockSpec((PP, D), lambda i: (0, 0)),
            pl.BlockSpec((PP, 8), lambda i: (0, 0)),
        ],
        out_specs=pl.BlockSpec((PP, 8), lambda i: (0, 0)),
        out_shape=jax.ShapeDtypeStruct((PP, 8), jnp.float32),
        scratch_shapes=[pltpu.VMEM((PP, D), jnp.float32)],
    )(Sq, ssq8, Ppad, psq8)

    pw, ord8, pi8 = pl.pallas_call(
        _tail_body,
        in_specs=[
            pl.BlockSpec((PP, 8), lambda: (0, 0)),
            pl.BlockSpec((P0, D), lambda: (0, 0)),
            pl.BlockSpec((D, D), lambda: (0, 0)),
        ],
        out_specs=(
            pl.BlockSpec((P0, D), lambda: (0, 0)),
            pl.BlockSpec((PP, 8), lambda: (0, 0)),
            pl.BlockSpec((PP, 8), lambda: (0, 0)),
        ),
        out_shape=(
            jax.ShapeDtypeStruct((P0, D), jnp.float32),
            jax.ShapeDtypeStruct((PP, 8), jnp.int32),
            jax.ShapeDtypeStruct((PP, 8), jnp.float32),
        ),
    )(counts_col, prototypes, W_proj)

    out2d = _make_sc_gather()(ord8[:, 0], prototypes, pw)
    return jnp.concatenate([pi8[:P0, 0], out2d.reshape(-1)])[None, :]


# speedup vs baseline: 4.7245x; 1.0130x over previous
import functools

import jax
import jax.numpy as jnp
from jax import lax
from jax.experimental import pallas as pl
from jax.experimental.pallas import tpu as pltpu
from jax.experimental.pallas import tpu_sc as plsc

P0 = 200
PP = 256
D = 1024
N = 16384
TOPK = 3
BLK = 2048
NEG = -3.0e38
RPW = 8
NW = P0 // RPW


def _counts_body(s_ref, ssq_ref, p_ref, psq_ref, out_ref, pn_ref):
    i = pl.program_id(0)

    @pl.when(i == 0)
    def _():
        pr = p_ref[...]
        pn_ref[...] = pr / (jnp.sqrt(psq_ref[...][:, 0:1]) + 1e-8)

    x = s_ref[...]
    xn = x / (jnp.sqrt(ssq_ref[...][:, 0:1]) + 1e-8)
    pn = pn_ref[...]
    sim = lax.dot_general(pn, xn, (((1,), (1,)), ((), ())),
                          preferred_element_type=jnp.float32)
    rows = lax.broadcasted_iota(jnp.int32, sim.shape, 0)
    m = jnp.where(rows < P0, sim, NEG)
    onehot = jnp.zeros_like(sim)
    for _ in range(TOPK):
        v = jnp.max(m, axis=0, keepdims=True)
        cand = jnp.where(m == v, rows, PP)
        imin = jnp.min(cand, axis=0, keepdims=True)
        sel = rows == imin
        onehot += sel.astype(jnp.float32)
        m = jnp.where(sel, NEG, m)
    blk = jnp.sum(onehot, axis=1, keepdims=True)

    @pl.when(i == 0)
    def _():
        out_ref[...] = jnp.zeros_like(out_ref)

    out_ref[...] += jnp.broadcast_to(blk, (PP, 8))


def _tail_body(c_ref, p_ref, w_ref, pw_ref, ord_ref, pi_ref):
    pw_ref[...] = jnp.dot(p_ref[...], w_ref[...],
                          preferred_element_type=jnp.float32)
    ccol = c_ref[...][:, 0:1]
    rows = lax.broadcasted_iota(jnp.int32, (PP, PP), 0)
    lanes = lax.broadcasted_iota(jnp.int32, (PP, PP), 1)
    diagd = jnp.where(rows == lanes, jnp.broadcast_to(ccol, (PP, PP)), 0.0)
    crow = lax.dot_general(jnp.ones((1, PP), jnp.float32), diagd,
                           (((1,), (0,)), ((), ())),
                           precision=lax.Precision.HIGHEST,
                           preferred_element_type=jnp.float32)
    ricol = lax.broadcasted_iota(jnp.int32, (PP, 1), 0)
    lirow = lax.broadcasted_iota(jnp.int32, (1, PP), 1)
    kcol = jnp.where(ricol < P0,
                     ccol.astype(jnp.int32) * PP + (PP - 1 - ricol),
                     -1 - ricol)
    krow = jnp.where(lirow < P0,
                     crow.astype(jnp.int32) * PP + (PP - 1 - lirow),
                     -1 - lirow)
    gt = (kcol > krow).astype(jnp.float32)
    rank = jnp.sum(gt, axis=0, keepdims=True)
    pm = (jnp.broadcast_to(rank.astype(jnp.int32), (PP, PP)) == rows)
    pmf = pm.astype(jnp.float32)
    ordv = jnp.sum(pmf * lanes.astype(jnp.float32), axis=1, keepdims=True)
    ord_i = jnp.minimum(ordv.astype(jnp.int32), P0 - 1)
    piv = jnp.sum(pmf * (crow / 49152.0), axis=1, keepdims=True)
    ord_ref[...] = jnp.broadcast_to(ord_i, (PP, 8))
    pi_ref[...] = jnp.broadcast_to(piv, (PP, 8))


@functools.lru_cache(maxsize=1)
def _make_sc_gather():
  @functools.partial(
      pl.kernel,
      mesh=plsc.VectorSubcoreMesh(core_axis_name="c", subcore_axis_name="s"),
      out_type=jax.ShapeDtypeStruct((P0, 2 * D), jnp.float32),
      scratch_types=[
          pltpu.VMEM((RPW,), jnp.int32),
          pltpu.VMEM((RPW, D), jnp.float32),
          pltpu.VMEM((RPW, D), jnp.float32),
          pltpu.SemaphoreType.DMA,
      ],
  )
  def _sc_gather(ord_hbm, proto_hbm, pw_hbm, out_hbm,
                 idx8, rows_a, rows_b, sem):
    wid = lax.axis_index("s") * 2 + lax.axis_index("c")

    @pl.when(wid < NW)
    def _():
      base = wid * RPW
      pltpu.sync_copy(ord_hbm.at[pl.ds(base, RPW)], idx8)
      ca = pltpu.async_copy(proto_hbm.at[idx8], rows_a, sem)
      cb = pltpu.async_copy(pw_hbm.at[idx8], rows_b, sem)
      ca.wait()
      cb.wait()
      pltpu.sync_copy(rows_a, out_hbm.at[pl.ds(base, RPW), pl.ds(0, D)])
      pltpu.sync_copy(rows_b, out_hbm.at[pl.ds(base, RPW), pl.ds(D, D)])

  return _sc_gather


def kernel(S, prototypes, W_proj):
    Sq = S[0]
    Ppad = jnp.pad(prototypes, ((0, PP - P0), (0, 0)))
    ssq8 = jnp.broadcast_to(jnp.sum(Sq * Sq, axis=-1, keepdims=True), (N, 8))
    psq8 = jnp.broadcast_to(
        jnp.sum(Ppad * Ppad, axis=-1, keepdims=True), (PP, 8))

    counts_col = pl.pallas_call(
        _counts_body,
        grid=(N // BLK,),
        in_specs=[
            pl.BlockSpec((BLK, D), lambda i: (i, 0)),
            pl.BlockSpec((BLK, 8), lambda i: (i, 0)),
            pl.BlockSpec((PP, D), lambda i: (0, 0)),
            pl.BlockSpec((PP, 8), lambda i: (0, 0)),
        ],
        out_specs=pl.BlockSpec((PP, 8), lambda i: (0, 0)),
        out_shape=jax.ShapeDtypeStruct((PP, 8), jnp.float32),
        scratch_shapes=[pltpu.VMEM((PP, D), jnp.float32)],
    )(Sq, ssq8, Ppad, psq8)

    pw, ord8, pi8 = pl.pallas_call(
        _tail_body,
        in_specs=[
            pl.BlockSpec((PP, 8), lambda: (0, 0)),
            pl.BlockSpec((P0, D), lambda: (0, 0)),
            pl.BlockSpec((D, D), lambda: (0, 0)),
        ],
        out_specs=(
            pl.BlockSpec((P0, D), lambda: (0, 0)),
            pl.BlockSpec((PP, 8), lambda: (0, 0)),
            pl.BlockSpec((PP, 8), lambda: (0, 0)),
        ),
        out_shape=(
            jax.ShapeDtypeStruct((P0, D), jnp.float32),
            jax.ShapeDtypeStruct((PP, 8), jnp.int32),
            jax.ShapeDtypeStruct((PP, 8), jnp.float32),
        ),
    )(counts_col, prototypes, W_proj)

    out2d = _make_sc_gather()(ord8[:, 0], prototypes, pw)
    return jnp.concatenate([pi8[:P0, 0], out2d.reshape(-1)])[None, :]
